# baseline (speedup 1.0000x reference)
import functools

import jax
import jax.numpy as jnp
from jax import lax
from jax.experimental import pallas as pl
from jax.experimental.pallas import tpu as pltpu
from jax.experimental.pallas import tpu_sc as plsc

_K = 8192
_D = 32
_N = 16384
_TB = 256
_COMMIT = 0.25
_NW = 32
_RPW = _N // _NW


def _vq_body(x_ref, wb_ref, x2_ref, w2_ref, idx_ref, q_ref, loss_ref,
             perp_ref, counts_ref, loss_acc_ref):
    i = pl.program_id(0)
    nsteps = pl.num_programs(0)

    x = x_ref[...]
    wb = wb_ref[...]

    ab = lax.dot_general(x.astype(jnp.bfloat16), wb, (((1,), (1,)), ((), ())),
                         preferred_element_type=jnp.float32)
    dist = x2_ref[...] + w2_ref[...] - 2.0 * ab

    half = _K // 2
    d0 = dist[:, :half]
    d1 = dist[:, half:]
    m0 = jnp.min(d0, axis=1)
    i0 = jnp.argmin(d0, axis=1)
    m1 = jnp.min(d1, axis=1)
    i1 = jnp.argmin(d1, axis=1) + half
    m0r = m0.astype(jnp.bfloat16).astype(jnp.float32)
    idx = jnp.where(m1 < m0r, i1, i0)
    idx_ref[...] = idx[:, None]
    cols = lax.broadcasted_iota(jnp.int32, (_TB, _K), 1)
    enc = jnp.where(cols == idx[:, None], 1.0, 0.0).astype(jnp.float32)

    enc_b = enc.astype(jnp.bfloat16)
    q = lax.dot_general(enc_b, wb, (((1,), (0,)), ((), ())),
                        preferred_element_type=jnp.float32)
    q_ref[...] = x + (q - x)

    @pl.when(i == 0)
    def _init():
        counts_ref[...] = jnp.zeros_like(counts_ref)
        loss_acc_ref[0] = 0.0

    ones_row = jnp.ones((1, _TB), jnp.bfloat16)
    counts_ref[...] += lax.dot_general(ones_row, enc_b, (((1,), (0,)), ((), ())),
                                       preferred_element_type=jnp.float32)
    loss_acc_ref[0] += jnp.sum((q - x) ** 2)

    @pl.when(i == nsteps - 1)
    def _finish():
        loss_ref[0] = _COMMIT * loss_acc_ref[0] / (_N * _D)
        p = counts_ref[...] / _N
        perp_ref[0] = jnp.exp(-jnp.sum(p * jnp.log(p + 1e-10)))


def _sc_onehot_body(idx_hbm, out_hbm, idx_v, buf_v):
    wid = lax.axis_index("s") * 2 + lax.axis_index("c")
    base = wid * _RPW
    pltpu.sync_copy(idx_hbm.at[pl.ds(base, _RPW)], idx_v)

    lane = lax.iota(jnp.int32, 16)
    ones16 = jnp.ones((16,), jnp.float32)
    zeros16 = jnp.zeros((16,), jnp.float32)

    def zrow(r):
        def zcol(j, c):
            buf_v[r, pl.ds(j * 16, 16)] = zeros16
            return c
        lax.fori_loop(0, _K // 16, zcol, 0)
    for r in range(8):
        zrow(r)

    lo_mask = lane < 8
    hi_mask = jnp.logical_not(lo_mask)
    lane_hi = lane - 8

    def body(g, c):
        colv = idx_v[pl.ds(g * 16, 16)]
        row0 = base + g * 16
        plsc.store_scatter(buf_v, [lane, colv], ones16, mask=lo_mask)
        pltpu.sync_copy(buf_v, out_hbm.at[pl.ds(row0, 8)])
        plsc.store_scatter(buf_v, [lane, colv], zeros16, mask=lo_mask)
        plsc.store_scatter(buf_v, [lane_hi, colv], ones16, mask=hi_mask)
        pltpu.sync_copy(buf_v, out_hbm.at[pl.ds(row0 + 8, 8)])
        plsc.store_scatter(buf_v, [lane_hi, colv], zeros16, mask=hi_mask)
        return c

    lax.fori_loop(0, _RPW // 16, body, 0)


@functools.partial(jax.jit, static_argnames=())
def _vq_fused(flat_x, w):
    grid = (_N // _TB,)
    pl_out = pl.pallas_call(
        _vq_body,
        grid=grid,
        in_specs=[
            pl.BlockSpec((_TB, _D), lambda i: (i, 0)),
            pl.BlockSpec((_K, _D), lambda i: (0, 0)),
            pl.BlockSpec((_TB, 1), lambda i: (i, 0)),
            pl.BlockSpec((1, _K), lambda i: (0, 0)),
        ],
        out_specs=[
            pl.BlockSpec((_TB, 1), lambda i: (i, 0)),
            pl.BlockSpec((_TB, _D), lambda i: (i, 0)),
            pl.BlockSpec(memory_space=pltpu.SMEM),
            pl.BlockSpec(memory_space=pltpu.SMEM),
        ],
        out_shape=[
            jax.ShapeDtypeStruct((_N, 1), jnp.int32),
            jax.ShapeDtypeStruct((_N, _D), jnp.float32),
            jax.ShapeDtypeStruct((1,), jnp.float32),
            jax.ShapeDtypeStruct((1,), jnp.float32),
        ],
        scratch_shapes=[
            pltpu.VMEM((1, _K), jnp.float32),
            pltpu.SMEM((1,), jnp.float32),
        ],
    )
    idx, q, loss, perp = pl_out(
        flat_x, w.astype(jnp.bfloat16),
        jnp.sum(flat_x ** 2, axis=1, keepdims=True),
        jnp.sum(w ** 2, axis=1)[None, :])

    sc_onehot = pl.kernel(
        _sc_onehot_body,
        out_type=jax.ShapeDtypeStruct((_N, _K), jnp.float32),
        mesh=plsc.VectorSubcoreMesh(core_axis_name="c", subcore_axis_name="s"),
        compiler_params=pltpu.CompilerParams(needs_layout_passes=False),
        scratch_types=[
            pltpu.VMEM((_RPW,), jnp.int32),
            pltpu.VMEM((8, _K), jnp.float32),
        ],
    )
    enc = sc_onehot(idx.reshape(_N))
    return enc, q, loss, perp


def kernel(inputs, embedding_weight):
    input_shape = inputs.shape
    flat_x = inputs.reshape(-1, _D)
    enc, q, loss, perp = _vq_fused(flat_x, embedding_weight)
    return (loss.reshape(()), q.reshape(input_shape), perp.reshape(()), enc)
